# baseline (speedup 1.0000x reference)
import math

import jax
import jax.numpy as jnp
import numpy as np
from jax.experimental import pallas as pl
from jax.experimental.pallas import tpu as pltpu

DIM = 512
QS = 8
QT = 256
KS = 16
KT = 256
TOPK = 32
WSTEPS = 5
WCOLS = WSTEPS * KT
NEG_INF = float("-inf")


def _proj_kv_body(src_ref, wk_ref, bk_ref, wv_ref, bv_ref, k_ref, v_ref):
    x = src_ref[0]
    k_ref[0] = jnp.dot(x, wk_ref[...], preferred_element_type=jnp.float32) + bk_ref[...]
    v_ref[0] = (jnp.dot(x.astype(jnp.bfloat16), wv_ref[...].astype(jnp.bfloat16),
                        preferred_element_type=jnp.float32) + bv_ref[...]).astype(jnp.bfloat16)


def _cond_body(ctx_ref, wc_ref, bc_ref, cond_ref):
    cond_ref[...] = jnp.dot(ctx_ref[...], wc_ref[...],
                            preferred_element_type=jnp.float32) + bc_ref[...]


def _band_start(t):
    c = (2 * (KS - 1) * t + (QS - 1)) // (2 * (QS - 1))
    s = jnp.clip(c - 2, 0, KS - WSTEPS)
    return s * KT


def _attend_body(q_ref, gamma_ref, beta_ref, wq_ref, bq_ref,
                 k_ref, v_ref, mask_ref, wo_ref, bo_ref, out_ref,
                 kv_ref, vv_ref, sem_k, sem_v,
                 s0_ref, s1_ref, s2_ref, s3_ref):
    b = pl.program_id(0)

    @pl.when(pl.program_id(1) == 0)
    def _load_kv():
        ck = pltpu.make_async_copy(k_ref.at[b], kv_ref, sem_k)
        cv = pltpu.make_async_copy(v_ref.at[b], vv_ref, sem_v)
        ck.start()
        cv.start()
        ck.wait()
        cv.wait()
    rows = jax.lax.broadcasted_iota(jnp.int32, gamma_ref.shape, 0)
    gamma = jnp.sum(jnp.where(rows == b, gamma_ref[...], 0.0), axis=0, keepdims=True)
    beta = jnp.sum(jnp.where(rows == b, beta_ref[...], 0.0), axis=0, keepdims=True)

    q = q_ref[0, 0]
    qm = q * (1.0 + gamma) + beta
    qp = jnp.dot(qm, wq_ref[...], preferred_element_type=jnp.float32) + bq_ref[...]

    start = _band_start(pl.program_id(1))
    kb = kv_ref[pl.ds(start, WCOLS), :]
    scores = jax.lax.dot_general(
        kb, qp, (((1,), (1,)), ((), ())), preferred_element_type=jnp.float32)
    scores = scores * (1.0 / math.sqrt(DIM)) + mask_ref[0]

    G = WCOLS // 4
    c0 = scores[0 * G:1 * G, :]
    c1 = scores[1 * G:2 * G, :]
    c2 = scores[2 * G:3 * G, :]
    c3 = scores[3 * G:4 * G, :]
    h0, l0 = jnp.maximum(c0, c1), jnp.minimum(c0, c1)
    h1, l1 = jnp.maximum(c2, c3), jnp.minimum(c2, c3)
    a0, a2 = jnp.maximum(h0, h1), jnp.minimum(h0, h1)
    a1, a3 = jnp.maximum(l0, l1), jnp.minimum(l0, l1)
    b1, b2 = jnp.maximum(a2, a1), jnp.minimum(a2, a1)
    s0_ref[...] = a0
    s1_ref[...] = b1
    s2_ref[...] = b2
    s3_ref[...] = a3

    row_max = jnp.max(a0, axis=0, keepdims=True)

    def body(_, m_prev):
        cur = s0_ref[...]
        hit = cur == m_prev
        nxt = jnp.where(hit, s1_ref[...], cur)
        s0_ref[...] = nxt
        s1_ref[...] = jnp.where(hit, s2_ref[...], s1_ref[...])
        s2_ref[...] = jnp.where(hit, s3_ref[...], s2_ref[...])
        s3_ref[...] = jnp.where(hit, NEG_INF, s3_ref[...])
        return jnp.max(nxt, axis=0, keepdims=True)

    thresh = jax.lax.fori_loop(0, TOPK - 1, body, row_max)

    w = jnp.where(scores >= thresh, jnp.exp(scores - row_max), 0.0).astype(jnp.bfloat16)
    vb = vv_ref[pl.ds(start, WCOLS), :]
    r_un = jax.lax.dot_general(
        w, vb, (((0,), (0,)), ((), ())), preferred_element_type=jnp.float32)
    z = jax.lax.dot_general(
        w, jnp.ones((WCOLS, 1), jnp.bfloat16), (((0,), (0,)), ((), ())),
        preferred_element_type=jnp.float32)
    r = r_un / z
    out_ref[0, 0] = jnp.dot(r.astype(jnp.bfloat16), wo_ref[...].astype(jnp.bfloat16),
                            preferred_element_type=jnp.float32) + bo_ref[...]


def kernel(query, source, contexts_0, contexts_1, Wq, bq, Wk, bk, Wv, bv, Wo, bo, Wc, bc, mask):
    bsz = query.shape[0]
    src_flat = source.reshape(bsz, KS * KT, DIM)
    ctx = jnp.concatenate([contexts_0, contexts_1], axis=-1)

    centers = np.round(np.linspace(0, KS - 1, QS)).astype(np.int64)
    starts = np.clip(centers - 2, 0, KS - WSTEPS) * KT
    mask_band = jnp.stack(
        [jax.lax.dynamic_slice(mask[t * QT], (int(starts[t]),), (WCOLS,))
         for t in range(QS)]).reshape(QS, WCOLS, 1)

    kp, vp = pl.pallas_call(
        _proj_kv_body,
        grid=(bsz, KS),
        in_specs=[
            pl.BlockSpec((1, KT, DIM), lambda b, s: (b, s, 0)),
            pl.BlockSpec((DIM, DIM), lambda b, s: (0, 0)),
            pl.BlockSpec((DIM,), lambda b, s: (0,)),
            pl.BlockSpec((DIM, DIM), lambda b, s: (0, 0)),
            pl.BlockSpec((DIM,), lambda b, s: (0,)),
        ],
        out_specs=[
            pl.BlockSpec((1, KT, DIM), lambda b, s: (b, s, 0)),
            pl.BlockSpec((1, KT, DIM), lambda b, s: (b, s, 0)),
        ],
        out_shape=[
            jax.ShapeDtypeStruct((bsz, KS * KT, DIM), jnp.float32),
            jax.ShapeDtypeStruct((bsz, KS * KT, DIM), jnp.bfloat16),
        ],
    )(src_flat, Wk, bk, Wv, bv)

    cond = pl.pallas_call(
        _cond_body,
        in_specs=[
            pl.BlockSpec((bsz, 2 * DIM), lambda: (0, 0)),
            pl.BlockSpec((2 * DIM, 3 * DIM), lambda: (0, 0)),
            pl.BlockSpec((3 * DIM,), lambda: (0,)),
        ],
        out_specs=pl.BlockSpec((bsz, 3 * DIM), lambda: (0, 0)),
        out_shape=jax.ShapeDtypeStruct((bsz, 3 * DIM), jnp.float32),
    )(ctx, Wc, bc)
    gamma = cond[:, :DIM]
    beta = cond[:, DIM:2 * DIM]

    out = pl.pallas_call(
        _attend_body,
        grid=(bsz, QS),
        in_specs=[
            pl.BlockSpec((1, 1, QT, DIM), lambda b, t: (b, t, 0, 0)),
            pl.BlockSpec((bsz, DIM), lambda b, t: (0, 0)),
            pl.BlockSpec((bsz, DIM), lambda b, t: (0, 0)),
            pl.BlockSpec((DIM, DIM), lambda b, t: (0, 0)),
            pl.BlockSpec((DIM,), lambda b, t: (0,)),
            pl.BlockSpec(memory_space=pl.ANY),
            pl.BlockSpec(memory_space=pl.ANY),
            pl.BlockSpec((1, WCOLS, 1), lambda b, t: (t, 0, 0)),
            pl.BlockSpec((DIM, DIM), lambda b, t: (0, 0)),
            pl.BlockSpec((DIM,), lambda b, t: (0,)),
        ],
        out_specs=pl.BlockSpec((1, 1, QT, DIM), lambda b, t: (b, t, 0, 0)),
        out_shape=jax.ShapeDtypeStruct((bsz, QS, QT, DIM), jnp.float32),
        scratch_shapes=[
            pltpu.VMEM((KS * KT, DIM), jnp.float32),
            pltpu.VMEM((KS * KT, DIM), jnp.bfloat16),
            pltpu.SemaphoreType.DMA,
            pltpu.SemaphoreType.DMA,
        ] + [pltpu.VMEM((WCOLS // 4, QT), jnp.float32)] * 4,
    )(query, gamma, beta, Wq, bq, kp, vp, mask_band, Wo, bo)

    return out


# speedup vs baseline: 1.0423x; 1.0423x over previous
import math

import jax
import jax.numpy as jnp
import numpy as np
from jax.experimental import pallas as pl
from jax.experimental.pallas import tpu as pltpu

DIM = 512
QS = 8
QT = 256
KS = 16
KT = 256
TOPK = 32
WSTEPS = 5
WCOLS = WSTEPS * KT
NEG_INF = float("-inf")


def _proj_kv_body(src_ref, wk_ref, bk_ref, wv_ref, bv_ref, k_ref, v_ref):
    x = src_ref[0]
    k_ref[0] = jnp.dot(x, wk_ref[...], preferred_element_type=jnp.float32) + bk_ref[...]
    v_ref[0] = (jnp.dot(x.astype(jnp.bfloat16), wv_ref[...].astype(jnp.bfloat16),
                        preferred_element_type=jnp.float32) + bv_ref[...]).astype(jnp.bfloat16)


def _cond_body(ctx_ref, wc_ref, bc_ref, cond_ref):
    cond_ref[...] = jnp.dot(ctx_ref[...], wc_ref[...],
                            preferred_element_type=jnp.float32) + bc_ref[...]


def _band_start(t):
    c = (2 * (KS - 1) * t + (QS - 1)) // (2 * (QS - 1))
    s = jnp.clip(c - 2, 0, KS - WSTEPS)
    return s * KT


def _attend_body(q_ref, gamma_ref, beta_ref, wq_ref, bq_ref,
                 k_ref, v_ref, mask_ref, wo_ref, bo_ref, out_ref,
                 s0_ref, s1_ref, s2_ref, s3_ref, ha_ref, hb_ref):
    b = pl.program_id(0)
    rows = jax.lax.broadcasted_iota(jnp.int32, gamma_ref.shape, 0)
    gamma = jnp.sum(jnp.where(rows == b, gamma_ref[...], 0.0), axis=0, keepdims=True)
    beta = jnp.sum(jnp.where(rows == b, beta_ref[...], 0.0), axis=0, keepdims=True)

    q = q_ref[0, 0]
    qm = q * (1.0 + gamma) + beta
    qp = jnp.dot(qm, wq_ref[...], preferred_element_type=jnp.float32) + bq_ref[...]

    start = _band_start(pl.program_id(1))
    kb = k_ref[0, pl.ds(start, WCOLS), :]
    scores = jax.lax.dot_general(
        kb, qp, (((1,), (1,)), ((), ())), preferred_element_type=jnp.float32)
    scores = scores * (1.0 / math.sqrt(DIM)) + mask_ref[0]

    G = WCOLS // 4
    cA0 = scores[0 * G:1 * G, :]
    cA1 = scores[1 * G:2 * G, :]
    cB0 = scores[2 * G:3 * G, :]
    cB1 = scores[3 * G:4 * G, :]
    s0_ref[...] = jnp.maximum(cA0, cA1)
    s1_ref[...] = jnp.minimum(cA0, cA1)
    s2_ref[...] = jnp.maximum(cB0, cB1)
    s3_ref[...] = jnp.minimum(cB0, cB1)

    mA0 = jnp.max(s0_ref[...], axis=0, keepdims=True)
    mB0 = jnp.max(s2_ref[...], axis=0, keepdims=True)
    ha_ref[0:1, :] = mA0
    hb_ref[0:1, :] = mB0
    row_max = jnp.maximum(mA0, mB0)

    def body(i, carry):
        mA, mB = carry
        curA = s0_ref[...]
        hitA = curA == mA
        nxtA = jnp.where(hitA, s1_ref[...], curA)
        s0_ref[...] = nxtA
        s1_ref[...] = jnp.where(hitA, NEG_INF, s1_ref[...])
        mA2 = jnp.max(nxtA, axis=0, keepdims=True)
        curB = s2_ref[...]
        hitB = curB == mB
        nxtB = jnp.where(hitB, s3_ref[...], curB)
        s2_ref[...] = nxtB
        s3_ref[...] = jnp.where(hitB, NEG_INF, s3_ref[...])
        mB2 = jnp.max(nxtB, axis=0, keepdims=True)
        ha_ref[pl.ds(i + 1, 1), :] = mA2
        hb_ref[pl.ds(i + 1, 1), :] = mB2
        return mA2, mB2

    jax.lax.fori_loop(0, TOPK - 1, body, (mA0, mB0))

    ha = ha_ref[...]
    hb = hb_ref[...]
    thresh = jnp.maximum(ha[TOPK - 1:TOPK, :], hb[TOPK - 1:TOPK, :])
    for j in range(TOPK - 1):
        thresh = jnp.maximum(
            thresh,
            jnp.minimum(ha[j:j + 1, :], hb[TOPK - 2 - j:TOPK - 1 - j, :]))

    w = jnp.where(scores >= thresh, jnp.exp(scores - row_max), 0.0).astype(jnp.bfloat16)
    vb = v_ref[0, pl.ds(start, WCOLS), :]
    r_un = jax.lax.dot_general(
        w, vb, (((0,), (0,)), ((), ())), preferred_element_type=jnp.float32)
    z = jax.lax.dot_general(
        w, jnp.ones((WCOLS, 1), jnp.bfloat16), (((0,), (0,)), ((), ())),
        preferred_element_type=jnp.float32)
    r = r_un / z
    out_ref[0, 0] = jnp.dot(r.astype(jnp.bfloat16), wo_ref[...].astype(jnp.bfloat16),
                            preferred_element_type=jnp.float32) + bo_ref[...]


def kernel(query, source, contexts_0, contexts_1, Wq, bq, Wk, bk, Wv, bv, Wo, bo, Wc, bc, mask):
    bsz = query.shape[0]
    src_flat = source.reshape(bsz, KS * KT, DIM)
    ctx = jnp.concatenate([contexts_0, contexts_1], axis=-1)

    centers = np.round(np.linspace(0, KS - 1, QS)).astype(np.int64)
    starts = np.clip(centers - 2, 0, KS - WSTEPS) * KT
    mask_band = jnp.stack(
        [jax.lax.dynamic_slice(mask[t * QT], (int(starts[t]),), (WCOLS,))
         for t in range(QS)]).reshape(QS, WCOLS, 1)

    kp, vp = pl.pallas_call(
        _proj_kv_body,
        grid=(bsz, KS),
        in_specs=[
            pl.BlockSpec((1, KT, DIM), lambda b, s: (b, s, 0)),
            pl.BlockSpec((DIM, DIM), lambda b, s: (0, 0)),
            pl.BlockSpec((DIM,), lambda b, s: (0,)),
            pl.BlockSpec((DIM, DIM), lambda b, s: (0, 0)),
            pl.BlockSpec((DIM,), lambda b, s: (0,)),
        ],
        out_specs=[
            pl.BlockSpec((1, KT, DIM), lambda b, s: (b, s, 0)),
            pl.BlockSpec((1, KT, DIM), lambda b, s: (b, s, 0)),
        ],
        out_shape=[
            jax.ShapeDtypeStruct((bsz, KS * KT, DIM), jnp.float32),
            jax.ShapeDtypeStruct((bsz, KS * KT, DIM), jnp.bfloat16),
        ],
    )(src_flat, Wk, bk, Wv, bv)

    cond = pl.pallas_call(
        _cond_body,
        in_specs=[
            pl.BlockSpec((bsz, 2 * DIM), lambda: (0, 0)),
            pl.BlockSpec((2 * DIM, 3 * DIM), lambda: (0, 0)),
            pl.BlockSpec((3 * DIM,), lambda: (0,)),
        ],
        out_specs=pl.BlockSpec((bsz, 3 * DIM), lambda: (0, 0)),
        out_shape=jax.ShapeDtypeStruct((bsz, 3 * DIM), jnp.float32),
    )(ctx, Wc, bc)
    gamma = cond[:, :DIM]
    beta = cond[:, DIM:2 * DIM]

    out = pl.pallas_call(
        _attend_body,
        grid=(bsz, QS),
        in_specs=[
            pl.BlockSpec((1, 1, QT, DIM), lambda b, t: (b, t, 0, 0)),
            pl.BlockSpec((bsz, DIM), lambda b, t: (0, 0)),
            pl.BlockSpec((bsz, DIM), lambda b, t: (0, 0)),
            pl.BlockSpec((DIM, DIM), lambda b, t: (0, 0)),
            pl.BlockSpec((DIM,), lambda b, t: (0,)),
            pl.BlockSpec((1, KS * KT, DIM), lambda b, t: (b, 0, 0)),
            pl.BlockSpec((1, KS * KT, DIM), lambda b, t: (b, 0, 0)),
            pl.BlockSpec((1, WCOLS, 1), lambda b, t: (t, 0, 0)),
            pl.BlockSpec((DIM, DIM), lambda b, t: (0, 0)),
            pl.BlockSpec((DIM,), lambda b, t: (0,)),
        ],
        out_specs=pl.BlockSpec((1, 1, QT, DIM), lambda b, t: (b, t, 0, 0)),
        out_shape=jax.ShapeDtypeStruct((bsz, QS, QT, DIM), jnp.float32),
        scratch_shapes=[pltpu.VMEM((WCOLS // 4, QT), jnp.float32)] * 4
        + [pltpu.VMEM((TOPK, QT), jnp.float32)] * 2,
    )(query, gamma, beta, Wq, bq, kp, vp, mask_band, Wo, bo)

    return out
